# baseline (speedup 1.0000x reference)
import functools

import jax
import jax.numpy as jnp
from jax import lax
from jax.experimental import pallas as pl
from jax.experimental.pallas import tpu as pltpu
from jax.experimental.pallas import tpu_sc as plsc

VOCAB = 1000


@functools.partial(jax.jit, static_argnums=(2, 3, 4, 5))
def _sc_gather(table, flat_idx, batch, t_len, t_pad, n_workers):
    idx_per_w = batch * t_pad // n_workers
    chunks_per_w = batch // n_workers
    assert chunks_per_w % 2 == 0 and t_pad % 8 == 0

    mesh = plsc.VectorSubcoreMesh(core_axis_name="c", subcore_axis_name="s")

    @functools.partial(
        pl.kernel,
        mesh=mesh,
        out_type=jax.ShapeDtypeStruct((batch, t_len, VOCAB), jnp.float32),
        compiler_params=pltpu.CompilerParams(use_tc_tiling_on_sc=False),
        scratch_types=[
            pltpu.VMEM((idx_per_w,), jnp.int32),
            pltpu.VMEM((t_len, VOCAB), jnp.float32),
            pltpu.VMEM((t_len, VOCAB), jnp.float32),
            pltpu.SemaphoreType.DMA,
            pltpu.SemaphoreType.DMA,
            pltpu.SemaphoreType.DMA,
            pltpu.SemaphoreType.DMA,
        ],
    )
    def gather_kernel(table_hbm, idx_hbm, out_hbm, idx_v, obuf_a, obuf_b,
                      gsem_a, gsem_b, osem_a, osem_b):
        wid = lax.axis_index("s") * 2 + lax.axis_index("c")
        ibase = pl.multiple_of(wid * idx_per_w, 8)
        base_b = wid * chunks_per_w

        pltpu.sync_copy(idx_hbm.at[pl.ds(ibase, idx_per_w)], idx_v)

        obufs = (obuf_a, obuf_b)
        gsems = (gsem_a, gsem_b)
        osems = (osem_a, osem_b)

        def chunk_body(c, carry):
            ids = idx_v.at[pl.ds(c * t_pad, t_len)]
            dg = pltpu.make_async_copy(table_hbm.at[ids], obufs[0], gsems[0])
            dg.start()
            dg.wait()
            do = pltpu.make_async_copy(obufs[0], out_hbm.at[base_b + c],
                                       osems[0])
            do.start()
            do.wait()
            return carry

        lax.fori_loop(0, chunks_per_w, chunk_body, 0)

    return gather_kernel(table, flat_idx)


def kernel(idx, table):
    B, T = idx.shape
    t_pad = (T + 7) // 8 * 8
    idx_p = jnp.pad(idx.astype(jnp.int32), ((0, 0), (0, t_pad - T)))
    flat = idx_p.reshape(-1)
    info = plsc.get_sparse_core_info()
    n_workers = info.num_cores * info.num_subcores
    return _sc_gather(table, flat, B, T, t_pad, n_workers)


# speedup vs baseline: 1.0289x; 1.0289x over previous
import functools

import jax
import jax.numpy as jnp
from jax import lax
from jax.experimental import pallas as pl
from jax.experimental.pallas import tpu as pltpu
from jax.experimental.pallas import tpu_sc as plsc

VOCAB = 1000


@functools.partial(jax.jit, static_argnums=(2, 3, 4, 5))
def _sc_gather(table, flat_idx, batch, t_len, t_pad, n_workers):
    idx_per_w = batch * t_pad // n_workers
    chunks_per_w = batch // n_workers
    assert chunks_per_w % 2 == 0 and t_pad % 8 == 0

    mesh = plsc.VectorSubcoreMesh(core_axis_name="c", subcore_axis_name="s")

    @functools.partial(
        pl.kernel,
        mesh=mesh,
        out_type=jax.ShapeDtypeStruct((batch, t_len, VOCAB), jnp.float32),
        compiler_params=pltpu.CompilerParams(use_tc_tiling_on_sc=False),
        scratch_types=[
            pltpu.VMEM((idx_per_w,), jnp.int32),
            pltpu.VMEM((t_len, VOCAB), jnp.float32),
            pltpu.VMEM((t_len, VOCAB), jnp.float32),
            pltpu.SemaphoreType.DMA,
            pltpu.SemaphoreType.DMA,
            pltpu.SemaphoreType.DMA,
            pltpu.SemaphoreType.DMA,
        ],
    )
    def gather_kernel(table_hbm, idx_hbm, out_hbm, idx_v, obuf_a, obuf_b,
                      gsem_a, gsem_b, osem_a, osem_b):
        wid = lax.axis_index("s") * 2 + lax.axis_index("c")
        ibase = pl.multiple_of(wid * idx_per_w, 8)
        base_b = wid * chunks_per_w

        pltpu.sync_copy(idx_hbm.at[pl.ds(ibase, idx_per_w)], idx_v)

        obufs = (obuf_a, obuf_b)
        gsems = (gsem_a, gsem_b)
        osems = (osem_a, osem_b)

        def gather_desc(c, s):
            ids = idx_v.at[pl.ds(c * t_pad, t_len)]
            return pltpu.make_async_copy(table_hbm.at[ids], obufs[s],
                                         gsems[s])

        def out_desc(c, s):
            return pltpu.make_async_copy(obufs[s], out_hbm.at[base_b + c],
                                         osems[s])

        gather_desc(0, 0).start()
        gather_desc(1, 1).start()

        def pair_body(p, carry):
            c0 = p * 2
            for s in range(2):
                c = c0 + s
                gather_desc(c, s).wait()
                out_desc(c, s).start()
                out_desc(c, s).wait()
                gather_desc(c + 2, s).start()
            return carry

        lax.fori_loop(0, chunks_per_w // 2 - 1, pair_body, 0)

        for s in range(2):
            c = chunks_per_w - 2 + s
            gather_desc(c, s).wait()
            out_desc(c, s).start()
            out_desc(c, s).wait()

    return gather_kernel(table, flat_idx)


def kernel(idx, table):
    B, T = idx.shape
    t_pad = (T + 7) // 8 * 8
    idx_p = jnp.pad(idx.astype(jnp.int32), ((0, 0), (0, t_pad - T)))
    flat = idx_p.reshape(-1)
    info = plsc.get_sparse_core_info()
    n_workers = info.num_cores * info.num_subcores
    return _sc_gather(table, flat, B, T, t_pad, n_workers)
